# baseline (speedup 1.0000x reference)
import functools

import jax
import jax.numpy as jnp
from jax import lax
from jax.experimental import pallas as pl
from jax.experimental.pallas import tpu as pltpu
from jax.experimental.pallas import tpu_sc as plsc

EPS = 1e-15


def _nrm(x):
    return jnp.clip(jnp.sqrt(jnp.sum(x * x, axis=-1, keepdims=True)), EPS, 1e15)


def _artanh(x):
    x = jnp.clip(x, -1 + 1e-7, 1 - 1e-7)
    return 0.5 * (jnp.log(1 + x) - jnp.log(1 - x))


def _proj(x):
    norm = _nrm(x)
    maxnorm = 1.0 - 1e-5
    return jnp.where(norm > maxnorm, x / norm * maxnorm, x)


def _expmap0(u):
    u_norm = _nrm(u)
    return jnp.tanh(u_norm) * u / u_norm


def _logmap0(p):
    p_norm = _nrm(p)
    return p / p_norm * _artanh(p_norm)


def _mobius_add(x, y):
    x2 = jnp.sum(x * x, -1, keepdims=True)
    y2 = jnp.sum(y * y, -1, keepdims=True)
    xy = jnp.sum(x * y, -1, keepdims=True)
    num = (1 + 2 * xy + y2) * x + (1 - x2) * y
    denom = 1 + 2 * xy + x2 * y2
    return num / jnp.clip(denom, EPS, None)


def _mobius_matvec(w, x):
    x_norm = _nrm(x)
    mx = lax.dot_general(
        x, w, (((1,), (1,)), ((), ())),
        preferred_element_type=jnp.float32, precision=lax.Precision.HIGHEST)
    mx_norm = _nrm(mx)
    res = jnp.tanh(mx_norm / x_norm * _artanh(x_norm)) * mx / mx_norm
    zero_rows = jnp.all(mx == 0, axis=-1, keepdims=True)
    return jnp.where(zero_rows, jnp.zeros_like(res), res)


def _pre_agg(x, w, b):
    mv = _proj(_mobius_matvec(w, x))
    bias = _proj(_expmap0(b))
    hlin = _proj(_mobius_add(mv, bias))
    return _logmap0(hlin)


def _post_agg(agg):
    hagg = _proj(_expmap0(agg))
    xt2 = jax.nn.relu(_logmap0(hagg))
    return _proj(_expmap0(xt2))


def _k_pre0(h_ref, w_ref, b_ref, o_ref):
    x = _proj(_expmap0(h_ref[...]))
    o_ref[...] = _pre_agg(x, w_ref[...], b_ref[...])


def _k_mid(p_ref, w_ref, b_ref, o_ref):
    x = _post_agg(p_ref[0] + p_ref[1])
    o_ref[...] = _pre_agg(x, w_ref[...], b_ref[...])


def _k_out(p_ref, wout_ref, bout_ref, o_ref):
    x = _post_agg(p_ref[0] + p_ref[1])
    o_ref[...] = lax.dot_general(
        x, wout_ref[...], (((1,), (1,)), ((), ())),
        preferred_element_type=jnp.float32,
        precision=lax.Precision.HIGHEST) + bout_ref[...]


def _tc_pre0(h, w, b, bn):
    n, d = h.shape
    return pl.pallas_call(
        _k_pre0,
        out_shape=jax.ShapeDtypeStruct((n, d), jnp.float32),
        grid=(n // bn,),
        in_specs=[
            pl.BlockSpec((bn, d), lambda i: (i, 0)),
            pl.BlockSpec((d, d), lambda i: (0, 0)),
            pl.BlockSpec((1, d), lambda i: (0, 0)),
        ],
        out_specs=pl.BlockSpec((bn, d), lambda i: (i, 0)),
    )(h, w, b)


def _tc_mid(p, w, b, bn, n):
    d = p.shape[-1]
    return pl.pallas_call(
        _k_mid,
        out_shape=jax.ShapeDtypeStruct((n, d), jnp.float32),
        grid=(n // bn,),
        in_specs=[
            pl.BlockSpec((2, bn, d), lambda i: (0, i, 0)),
            pl.BlockSpec((d, d), lambda i: (0, 0)),
            pl.BlockSpec((1, d), lambda i: (0, 0)),
        ],
        out_specs=pl.BlockSpec((bn, d), lambda i: (i, 0)),
    )(p, w, b)


def _tc_out(p, wout, bout, bn, n):
    d = p.shape[-1]
    z = wout.shape[0]
    return pl.pallas_call(
        _k_out,
        out_shape=jax.ShapeDtypeStruct((n, z), jnp.float32),
        grid=(n // bn,),
        in_specs=[
            pl.BlockSpec((2, bn, d), lambda i: (0, i, 0)),
            pl.BlockSpec((z, d), lambda i: (0, 0)),
            pl.BlockSpec((1, z), lambda i: (0, 0)),
        ],
        out_specs=pl.BlockSpec((bn, z), lambda i: (i, 0)),
    )(p, wout, bout)


_CH = 128
_NC = 2
_NS = 16


_NBUF = 2
_NIDX = 4


@functools.cache
def _make_sc_agg(n, d, e_pad, n_pad):
    cpw = e_pad // (_NC * _NS * _CH)
    zch = n_pad // (_NS * _CH)
    outr = n_pad // _NS
    assert cpw % _NIDX == 0 and cpw >= 2 * _NIDX

    mesh = plsc.VectorSubcoreMesh(core_axis_name="c", subcore_axis_name="s")

    @functools.partial(
        pl.kernel,
        out_type=jax.ShapeDtypeStruct((_NC, n_pad, d), jnp.float32),
        mesh=mesh,
        scratch_types=[
            pltpu.VMEM_SHARED((n_pad, d), jnp.float32),
            [pltpu.VMEM((_CH,), jnp.int32)] * _NIDX,
            [pltpu.VMEM((_CH,), jnp.int32)] * _NIDX,
            [pltpu.VMEM((_CH, d), jnp.float32)] * _NBUF,
            [pltpu.SemaphoreType.DMA] * _NBUF,
            [pltpu.SemaphoreType.DMA] * _NBUF,
            [pltpu.SemaphoreType.DMA] * _NIDX,
            [pltpu.SemaphoreType.DMA] * _NIDX,
        ],
    )
    def agg_kernel(src_hbm, dst_hbm, xt_hbm, zeros_hbm, out_hbm,
                   acc_sh, idx_s, idx_d, rows, sem_g, sem_s, sem_is, sem_id):
        cid = lax.axis_index("c")
        sid = lax.axis_index("s")
        wid = cid * _NS + sid
        base = wid * cpw

        pltpu.sync_copy(zeros_hbm, rows[0])

        @pl.loop(0, zch)
        def _zero(k):
            pltpu.sync_copy(rows[0], acc_sh.at[pl.ds((sid * zch + k) * _CH, _CH)])

        plsc.subcore_barrier()

        def _idx(c, q):
            off = (base + c) * _CH
            pltpu.async_copy(src_hbm.at[pl.ds(off, _CH)], idx_s[q], sem_is[q])
            pltpu.async_copy(dst_hbm.at[pl.ds(off, _CH)], idx_d[q], sem_id[q])

        def _gather(b, q):
            pltpu.async_copy(xt_hbm.at[idx_s[q]], rows[b], sem_g[b])

        def _scatter(b, q):
            pltpu.async_copy(rows[b], acc_sh.at[idx_d[q]], sem_s[b], add=True)

        def _drain_rows(sem):
            pltpu.make_async_copy(xt_hbm.at[pl.ds(0, _CH)], rows[0], sem).wait()

        def _drain_idx(sem):
            pltpu.make_async_copy(src_hbm.at[pl.ds(0, _CH)], idx_s[0],
                                  sem).wait()

        def _step_full(c, j):
            b, bp = j % _NBUF, (j - 1) % _NBUF
            q, qp, qn = j % _NIDX, (j - 1) % _NIDX, (j + 2) % _NIDX
            _drain_rows(sem_s[b])
            _idx(c + 2, qn)
            _drain_idx(sem_is[q])
            _gather(b, q)
            _drain_rows(sem_g[bp])
            _drain_idx(sem_id[qp])
            _scatter(bp, qp)

        _idx(0, 0)
        _idx(1, 1)
        _idx(2, 2)
        _drain_idx(sem_is[0])
        _gather(0, 0)
        _idx(3, 3)
        _drain_idx(sem_is[1])
        _gather(1, 1)
        _drain_rows(sem_g[0])
        _drain_idx(sem_id[0])
        _scatter(0, 0)
        _step_full(2, 2)
        _step_full(3, 3)

        @pl.loop(1, cpw // _NIDX)
        def _edges(g):
            for j in range(_NIDX):
                _step_full(g * _NIDX + j, j)

        _drain_rows(sem_g[(cpw - 1) % _NBUF])
        _drain_idx(sem_id[(cpw - 1) % _NIDX])
        _scatter((cpw - 1) % _NBUF, (cpw - 1) % _NIDX)
        for b in range(_NBUF):
            _drain_rows(sem_s[b])
        for q in (cpw % _NIDX, (cpw + 1) % _NIDX):
            _drain_idx(sem_is[q])
            _drain_idx(sem_id[q])

        plsc.subcore_barrier()
        pltpu.sync_copy(acc_sh.at[pl.ds(sid * outr, outr)],
                        out_hbm.at[cid, pl.ds(sid * outr, outr)])

    return agg_kernel


def kernel(h, distances, edges, node_mask, edge_mask, W1, b1, W2, b2, Wout, bout):
    n, d = h.shape
    e = edges.shape[1]

    bn = 2000 if n % 2000 == 0 else n

    chunk_tot = _NC * _NS * _CH * _NBUF
    e_pad = -(-e // chunk_tot) * chunk_tot
    n_pad = -(-(n + 1) // (_NS * _CH)) * (_NS * _CH)

    src = edges[0].astype(jnp.int32)
    dst = edges[1].astype(jnp.int32)
    pad = e_pad - e + 2 * _CH
    src = jnp.concatenate([src, jnp.zeros((pad,), jnp.int32)])
    dst = jnp.concatenate([dst, jnp.full((pad,), n, jnp.int32)])
    zeros_in = jnp.zeros((_CH, d), jnp.float32)

    sc_agg = _make_sc_agg(n, d, e_pad, n_pad)

    b1r = b1.reshape(1, d)
    b2r = b2.reshape(1, d)
    boutr = bout.reshape(1, -1)

    xt = _tc_pre0(h, W1, b1r, bn)
    p = sc_agg(src, dst, xt, zeros_in)
    xt = _tc_mid(p, W2, b2r, bn, n)
    p = sc_agg(src, dst, xt, zeros_in)
    return _tc_out(p, Wout, boutr, bn, n)


# speedup vs baseline: 3.2018x; 3.2018x over previous
import functools

import jax
import jax.numpy as jnp
from jax import lax
from jax.experimental import pallas as pl
from jax.experimental.pallas import tpu as pltpu
from jax.experimental.pallas import tpu_sc as plsc

EPS = 1e-15


def _nrm(x):
    return jnp.clip(jnp.sqrt(jnp.sum(x * x, axis=-1, keepdims=True)), EPS, 1e15)


def _artanh(x):
    x = jnp.clip(x, -1 + 1e-7, 1 - 1e-7)
    return 0.5 * (jnp.log(1 + x) - jnp.log(1 - x))


def _proj(x):
    norm = _nrm(x)
    maxnorm = 1.0 - 1e-5
    return jnp.where(norm > maxnorm, x / norm * maxnorm, x)


def _expmap0(u):
    u_norm = _nrm(u)
    return jnp.tanh(u_norm) * u / u_norm


def _logmap0(p):
    p_norm = _nrm(p)
    return p / p_norm * _artanh(p_norm)


def _mobius_add(x, y):
    x2 = jnp.sum(x * x, -1, keepdims=True)
    y2 = jnp.sum(y * y, -1, keepdims=True)
    xy = jnp.sum(x * y, -1, keepdims=True)
    num = (1 + 2 * xy + y2) * x + (1 - x2) * y
    denom = 1 + 2 * xy + x2 * y2
    return num / jnp.clip(denom, EPS, None)


def _mobius_matvec(w, x):
    x_norm = _nrm(x)
    mx = lax.dot_general(
        x, w, (((1,), (1,)), ((), ())),
        preferred_element_type=jnp.float32, precision=lax.Precision.HIGHEST)
    mx_norm = _nrm(mx)
    res = jnp.tanh(mx_norm / x_norm * _artanh(x_norm)) * mx / mx_norm
    zero_rows = jnp.all(mx == 0, axis=-1, keepdims=True)
    return jnp.where(zero_rows, jnp.zeros_like(res), res)


def _pre_agg(x, w, b):
    mv = _proj(_mobius_matvec(w, x))
    bias = _proj(_expmap0(b))
    hlin = _proj(_mobius_add(mv, bias))
    return _logmap0(hlin)


def _post_agg(agg):
    hagg = _proj(_expmap0(agg))
    xt2 = jax.nn.relu(_logmap0(hagg))
    return _proj(_expmap0(xt2))


def _k_pre0(h_ref, w_ref, b_ref, o_ref):
    x = _proj(_expmap0(h_ref[...]))
    o_ref[...] = _pre_agg(x, w_ref[...], b_ref[...])


def _k_mid(p_ref, w_ref, b_ref, o_ref):
    x = _post_agg(p_ref[0] + p_ref[1])
    o_ref[...] = _pre_agg(x, w_ref[...], b_ref[...])


def _k_out(p_ref, wout_ref, bout_ref, o_ref):
    x = _post_agg(p_ref[0] + p_ref[1])
    o_ref[...] = lax.dot_general(
        x, wout_ref[...], (((1,), (1,)), ((), ())),
        preferred_element_type=jnp.float32,
        precision=lax.Precision.HIGHEST) + bout_ref[...]


def _tc_pre0(h, w, b, bn):
    n, d = h.shape
    return pl.pallas_call(
        _k_pre0,
        out_shape=jax.ShapeDtypeStruct((n, d), jnp.float32),
        grid=(n // bn,),
        in_specs=[
            pl.BlockSpec((bn, d), lambda i: (i, 0)),
            pl.BlockSpec((d, d), lambda i: (0, 0)),
            pl.BlockSpec((1, d), lambda i: (0, 0)),
        ],
        out_specs=pl.BlockSpec((bn, d), lambda i: (i, 0)),
    )(h, w, b)


def _tc_mid(p, w, b, bn, n):
    d = p.shape[-1]
    return pl.pallas_call(
        _k_mid,
        out_shape=jax.ShapeDtypeStruct((n, d), jnp.float32),
        grid=(n // bn,),
        in_specs=[
            pl.BlockSpec((2, bn, d), lambda i: (0, i, 0)),
            pl.BlockSpec((d, d), lambda i: (0, 0)),
            pl.BlockSpec((1, d), lambda i: (0, 0)),
        ],
        out_specs=pl.BlockSpec((bn, d), lambda i: (i, 0)),
    )(p, w, b)


def _tc_out(p, wout, bout, bn, n):
    d = p.shape[-1]
    z = wout.shape[0]
    return pl.pallas_call(
        _k_out,
        out_shape=jax.ShapeDtypeStruct((n, z), jnp.float32),
        grid=(n // bn,),
        in_specs=[
            pl.BlockSpec((2, bn, d), lambda i: (0, i, 0)),
            pl.BlockSpec((z, d), lambda i: (0, 0)),
            pl.BlockSpec((1, z), lambda i: (0, 0)),
        ],
        out_specs=pl.BlockSpec((bn, z), lambda i: (i, 0)),
    )(p, wout, bout)


_CH = 128
_NC = 2
_NS = 16


_NBUF = 2
_NIDX = 4


@functools.cache
def _make_sc_agg(n, d, e_pad, n_pad):
    cpw = e_pad // (_NC * _NS * _CH)
    zch = n_pad // (_NS * _CH)
    outr = n_pad // _NS
    assert cpw % _NIDX == 0 and cpw >= 2 * _NIDX

    mesh = plsc.VectorSubcoreMesh(core_axis_name="c", subcore_axis_name="s")

    @functools.partial(
        pl.kernel,
        out_type=jax.ShapeDtypeStruct((_NC, n_pad, d), jnp.float32),
        mesh=mesh,
        scratch_types=[
            pltpu.VMEM_SHARED((n_pad, d), jnp.float32),
            [pltpu.VMEM((_CH,), jnp.int32)] * _NIDX,
            [pltpu.VMEM((_CH,), jnp.int32)] * _NIDX,
            [pltpu.VMEM((_CH, d), jnp.float32)] * _NBUF,
            [pltpu.SemaphoreType.DMA] * _NBUF,
            [pltpu.SemaphoreType.DMA] * _NBUF,
            [pltpu.SemaphoreType.DMA] * _NIDX,
            [pltpu.SemaphoreType.DMA] * _NIDX,
        ],
    )
    def agg_kernel(src_hbm, dst_hbm, xt_hbm, zeros_hbm, out_hbm,
                   acc_sh, idx_s, idx_d, rows, sem_g, sem_s, sem_is, sem_id):
        cid = lax.axis_index("c")
        sid = lax.axis_index("s")
        wid = cid * _NS + sid

        pltpu.sync_copy(zeros_hbm, rows[0])

        @pl.loop(0, zch)
        def _zero(k):
            pltpu.sync_copy(rows[0], acc_sh.at[pl.ds((sid * zch + k) * _CH, _CH)])

        plsc.subcore_barrier()

        def _idx(c, q):
            off = (c * _NC * _NS + wid) * _CH
            pltpu.async_copy(src_hbm.at[pl.ds(off, _CH)], idx_s[q], sem_is[q])
            pltpu.async_copy(dst_hbm.at[pl.ds(off, _CH)], idx_d[q], sem_id[q])

        def _gather(b, q):
            pltpu.async_copy(xt_hbm.at[idx_s[q]], rows[b], sem_g[b])

        def _scatter(b, q):
            pltpu.async_copy(rows[b], acc_sh.at[idx_d[q]], sem_s[b], add=True)

        def _drain_rows(sem):
            pltpu.make_async_copy(xt_hbm.at[pl.ds(0, _CH)], rows[0], sem).wait()

        def _drain_idx(sem):
            pltpu.make_async_copy(src_hbm.at[pl.ds(0, _CH)], idx_s[0],
                                  sem).wait()

        def _step_full(c, j):
            b, bp = j % _NBUF, (j - 1) % _NBUF
            q, qp, qn = j % _NIDX, (j - 1) % _NIDX, (j + 2) % _NIDX
            _drain_rows(sem_s[b])
            _idx(c + 2, qn)
            _drain_idx(sem_is[q])
            _gather(b, q)
            _drain_rows(sem_g[bp])
            _drain_idx(sem_id[qp])
            _scatter(bp, qp)

        _idx(0, 0)
        _idx(1, 1)
        _idx(2, 2)
        _drain_idx(sem_is[0])
        _gather(0, 0)
        _idx(3, 3)
        _drain_idx(sem_is[1])
        _gather(1, 1)
        _drain_rows(sem_g[0])
        _drain_idx(sem_id[0])
        _scatter(0, 0)
        _step_full(2, 2)
        _step_full(3, 3)

        @pl.loop(1, cpw // _NIDX)
        def _edges(g):
            for j in range(_NIDX):
                _step_full(g * _NIDX + j, j)

        _drain_rows(sem_g[(cpw - 1) % _NBUF])
        _drain_idx(sem_id[(cpw - 1) % _NIDX])
        _scatter((cpw - 1) % _NBUF, (cpw - 1) % _NIDX)
        for b in range(_NBUF):
            _drain_rows(sem_s[b])
        for q in (cpw % _NIDX, (cpw + 1) % _NIDX):
            _drain_idx(sem_is[q])
            _drain_idx(sem_id[q])

        plsc.subcore_barrier()
        pltpu.sync_copy(acc_sh.at[pl.ds(sid * outr, outr)],
                        out_hbm.at[cid, pl.ds(sid * outr, outr)])

    return agg_kernel


def kernel(h, distances, edges, node_mask, edge_mask, W1, b1, W2, b2, Wout, bout):
    n, d = h.shape
    e = edges.shape[1]

    bn = 2000 if n % 2000 == 0 else n

    chunk_tot = _NC * _NS * _CH * _NBUF
    e_pad = -(-e // chunk_tot) * chunk_tot
    n_pad = -(-(n + 1) // (_NS * _CH)) * (_NS * _CH)

    src = edges[0].astype(jnp.int32)
    dst = edges[1].astype(jnp.int32)
    pad = e_pad - e + 2 * _NC * _NS * _CH
    pad_i = jnp.arange(pad, dtype=jnp.int32)
    n_trash = n_pad - n
    src = jnp.concatenate([src, pad_i % n])
    dst = jnp.concatenate([dst, n + pad_i % n_trash])
    zeros_in = jnp.zeros((_CH, d), jnp.float32)

    sc_agg = _make_sc_agg(n, d, e_pad, n_pad)

    b1r = b1.reshape(1, d)
    b2r = b2.reshape(1, d)
    boutr = bout.reshape(1, -1)

    xt = _tc_pre0(h, W1, b1r, bn)
    p = sc_agg(src, dst, xt, zeros_in)
    xt = _tc_mid(p, W2, b2r, bn, n)
    p = sc_agg(src, dst, xt, zeros_in)
    return _tc_out(p, Wout, boutr, bn, n)


# speedup vs baseline: 3.3871x; 1.0579x over previous
import functools

import numpy as np

import jax
import jax.numpy as jnp
from jax import lax
from jax.experimental import pallas as pl
from jax.experimental.pallas import tpu as pltpu
from jax.experimental.pallas import tpu_sc as plsc

EPS = 1e-15


_MAXN = 1.0 - 1e-5


def _rowss(x):
    return jnp.sum(x * x, axis=-1)


def _artanh(x):
    x = jnp.clip(x, -1 + 1e-7, 1 - 1e-7)
    return 0.5 * jnp.log((1 + x) / (1 - x))


def _expproj(ss):
    nu = jnp.clip(jnp.sqrt(ss), EPS, 1e15)
    t = jnp.tanh(nu)
    s = (t / nu) * jnp.minimum(1.0, _MAXN / jnp.clip(t, EPS, 1e15))
    nx = jnp.clip(jnp.minimum(t, _MAXN), EPS, 1e15)
    return s, nx


def _postagg_relu(agg):
    na = jnp.clip(jnp.sqrt(_rowss(agg)), EPS, 1e15)
    t = jnp.tanh(na)
    s_exp = t / na
    s_p = jnp.minimum(1.0, _MAXN / jnp.clip(t, EPS, 1e15))
    nh = jnp.clip(jnp.minimum(t, _MAXN), EPS, 1e15)
    s_lg = _artanh(nh) / nh
    return jax.nn.relu(agg * (s_exp * s_p * s_lg)[:, None])


def _bias_row(b):
    nb = jnp.clip(jnp.sqrt(jnp.sum(b * b)), EPS, 1e15)
    e = b * (jnp.tanh(nb) / nb)
    ne = jnp.clip(jnp.sqrt(jnp.sum(e * e)), EPS, 1e15)
    bias = e * jnp.minimum(1.0, _MAXN / ne)
    return bias, jnp.sum(bias * bias)


def _hyplin_logmap(u, sx, nx, w, bias, y2):
    mm = lax.dot_general(
        u, w, (((1,), (1,)), ((), ())),
        preferred_element_type=jnp.float32, precision=lax.Precision.HIGHEST)
    ssmm = _rowss(mm)
    nmx = jnp.clip(jnp.sqrt(ssmm) * sx, EPS, 1e15)
    r = nmx / nx * _artanh(nx)
    tr = jnp.tanh(r)
    zero = ssmm == 0
    n_res = jnp.where(zero, 0.0, tr)
    s_mv = jnp.where(zero, 0.0, tr / nmx) * sx
    s_pmv = jnp.minimum(1.0, _MAXN / jnp.clip(n_res, EPS, 1e15))
    nmv = jnp.minimum(n_res, _MAXN)
    x2 = nmv * nmv
    s_all = s_mv * s_pmv
    xy = jnp.sum(mm * bias, axis=-1) * s_all
    den = jnp.clip(1 + 2 * xy + x2 * y2, EPS, None)
    a = (1 + 2 * xy + y2) / den
    b2_ = (1 - x2) / den
    ssh = a * a * x2 + 2 * a * b2_ * xy + b2_ * b2_ * y2
    nhl = jnp.clip(jnp.sqrt(ssh), EPS, 1e15)
    s_ph = jnp.minimum(1.0, _MAXN / nhl)
    nhc = jnp.clip(jnp.minimum(nhl, _MAXN), EPS, 1e15)
    f = s_ph * (_artanh(nhc) / nhc)
    return mm * (s_all * a * f)[:, None] + bias * (b2_ * f)[:, None]


def _k_pre0(h_ref, w_ref, b_ref, o_ref):
    h = h_ref[...]
    sx, nx = _expproj(_rowss(h))
    bias, y2 = _bias_row(b_ref[...])
    o_ref[...] = _hyplin_logmap(h, sx, nx, w_ref[...], bias, y2)


def _k_mid(p_ref, w_ref, b_ref, o_ref):
    xt2 = _postagg_relu(p_ref[0] + p_ref[1])
    sx, nx = _expproj(_rowss(xt2))
    bias, y2 = _bias_row(b_ref[...])
    o_ref[...] = _hyplin_logmap(xt2, sx, nx, w_ref[...], bias, y2)


def _k_out(p_ref, wout_ref, bout_ref, o_ref):
    xt2 = _postagg_relu(p_ref[0] + p_ref[1])
    sx, _ = _expproj(_rowss(xt2))
    mm = lax.dot_general(
        xt2, wout_ref[...], (((1,), (1,)), ((), ())),
        preferred_element_type=jnp.float32, precision=lax.Precision.HIGHEST)
    o_ref[...] = mm * sx[:, None] + bout_ref[...]


def _tc_pre0(h, w, b, bn):
    n, d = h.shape
    return pl.pallas_call(
        _k_pre0,
        out_shape=jax.ShapeDtypeStruct((n, d), jnp.float32),
        grid=(n // bn,),
        in_specs=[
            pl.BlockSpec((bn, d), lambda i: (i, 0)),
            pl.BlockSpec((d, d), lambda i: (0, 0)),
            pl.BlockSpec((1, d), lambda i: (0, 0)),
        ],
        out_specs=pl.BlockSpec((bn, d), lambda i: (i, 0)),
    )(h, w, b)


def _tc_mid(p, w, b, bn, n):
    d = p.shape[-1]
    return pl.pallas_call(
        _k_mid,
        out_shape=jax.ShapeDtypeStruct((n, d), jnp.float32),
        grid=(n // bn,),
        in_specs=[
            pl.BlockSpec((2, bn, d), lambda i: (0, i, 0)),
            pl.BlockSpec((d, d), lambda i: (0, 0)),
            pl.BlockSpec((1, d), lambda i: (0, 0)),
        ],
        out_specs=pl.BlockSpec((bn, d), lambda i: (i, 0)),
    )(p, w, b)


def _tc_out(p, wout, bout, bn, n):
    d = p.shape[-1]
    z = wout.shape[0]
    return pl.pallas_call(
        _k_out,
        out_shape=jax.ShapeDtypeStruct((n, z), jnp.float32),
        grid=(n // bn,),
        in_specs=[
            pl.BlockSpec((2, bn, d), lambda i: (0, i, 0)),
            pl.BlockSpec((z, d), lambda i: (0, 0)),
            pl.BlockSpec((1, z), lambda i: (0, 0)),
        ],
        out_specs=pl.BlockSpec((bn, z), lambda i: (i, 0)),
    )(p, wout, bout)


_CH = 128
_NC = 2
_NS = 16


_NBUF = 2
_NIDX = 4


@functools.cache
def _make_sc_agg(n, d, e_pad, n_pad):
    cpw = e_pad // (_NC * _NS * _CH)
    zch = n_pad // (_NS * _CH)
    outr = n_pad // _NS
    assert cpw % _NIDX == 0 and cpw >= 2 * _NIDX

    mesh = plsc.VectorSubcoreMesh(core_axis_name="c", subcore_axis_name="s")

    @functools.partial(
        pl.kernel,
        out_type=jax.ShapeDtypeStruct((_NC, n_pad, d), jnp.float32),
        mesh=mesh,
        scratch_types=[
            pltpu.VMEM_SHARED((n_pad, d), jnp.float32),
            [pltpu.VMEM((_CH,), jnp.int32)] * _NIDX,
            [pltpu.VMEM((_CH,), jnp.int32)] * _NIDX,
            [pltpu.VMEM((_CH, d), jnp.float32)] * _NBUF,
            [pltpu.SemaphoreType.DMA] * _NBUF,
            [pltpu.SemaphoreType.DMA] * _NBUF,
            [pltpu.SemaphoreType.DMA] * _NIDX,
            [pltpu.SemaphoreType.DMA] * _NIDX,
        ],
    )
    def agg_kernel(src_hbm, dst_hbm, xt_hbm, zeros_hbm, out_hbm,
                   acc_sh, idx_s, idx_d, rows, sem_g, sem_s, sem_is, sem_id):
        cid = lax.axis_index("c")
        sid = lax.axis_index("s")
        wid = cid * _NS + sid

        pltpu.sync_copy(zeros_hbm, rows[0])

        @pl.loop(0, zch)
        def _zero(k):
            pltpu.sync_copy(rows[0], acc_sh.at[pl.ds((sid * zch + k) * _CH, _CH)])

        plsc.subcore_barrier()

        def _idx(c, q):
            off = (c * _NC * _NS + wid) * _CH
            pltpu.async_copy(src_hbm.at[pl.ds(off, _CH)], idx_s[q], sem_is[q])
            pltpu.async_copy(dst_hbm.at[pl.ds(off, _CH)], idx_d[q], sem_id[q])

        def _gather(b, q):
            pltpu.async_copy(xt_hbm.at[idx_s[q]], rows[b], sem_g[b])

        def _scatter(b, q):
            pltpu.async_copy(rows[b], acc_sh.at[idx_d[q]], sem_s[b], add=True)

        def _drain_rows(sem):
            pltpu.make_async_copy(xt_hbm.at[pl.ds(0, _CH)], rows[0], sem).wait()

        def _drain_idx(sem):
            pltpu.make_async_copy(src_hbm.at[pl.ds(0, _CH)], idx_s[0],
                                  sem).wait()

        def _step_full(c, j):
            b, bp = j % _NBUF, (j - 1) % _NBUF
            q, qp, qn = j % _NIDX, (j - 1) % _NIDX, (j + 2) % _NIDX
            _drain_rows(sem_s[b])
            _idx(c + 2, qn)
            _drain_idx(sem_is[q])
            _gather(b, q)
            _drain_rows(sem_g[bp])
            _drain_idx(sem_id[qp])
            _scatter(bp, qp)

        _idx(0, 0)
        _idx(1, 1)
        _idx(2, 2)
        _drain_idx(sem_is[0])
        _gather(0, 0)
        _idx(3, 3)
        _drain_idx(sem_is[1])
        _gather(1, 1)
        _drain_rows(sem_g[0])
        _drain_idx(sem_id[0])
        _scatter(0, 0)
        _step_full(2, 2)
        _step_full(3, 3)

        @pl.loop(1, cpw // _NIDX)
        def _edges(g):
            for j in range(_NIDX):
                _step_full(g * _NIDX + j, j)

        _drain_rows(sem_g[(cpw - 1) % _NBUF])
        _drain_idx(sem_id[(cpw - 1) % _NIDX])
        _scatter((cpw - 1) % _NBUF, (cpw - 1) % _NIDX)
        for b in range(_NBUF):
            _drain_rows(sem_s[b])
        for q in (cpw % _NIDX, (cpw + 1) % _NIDX):
            _drain_idx(sem_is[q])
            _drain_idx(sem_id[q])

        plsc.subcore_barrier()
        pltpu.sync_copy(acc_sh.at[pl.ds(sid * outr, outr)],
                        out_hbm.at[cid, pl.ds(sid * outr, outr)])

    return agg_kernel


def kernel(h, distances, edges, node_mask, edge_mask, W1, b1, W2, b2, Wout, bout):
    n, d = h.shape
    e = edges.shape[1]

    bn = 2000 if n % 2000 == 0 else n

    chunk_tot = _NC * _NS * _CH * _NBUF
    e_pad = -(-e // chunk_tot) * chunk_tot
    n_pad = -(-(n + 1) // (_NS * _CH)) * (_NS * _CH)

    src = edges[0].astype(jnp.int32)
    dst = edges[1].astype(jnp.int32)
    pad = e_pad - e + 2 * _NC * _NS * _CH
    pad_i = np.arange(pad, dtype=np.int32)
    n_trash = n_pad - n
    src = jnp.concatenate([src, jnp.asarray(pad_i % n)])
    dst = jnp.concatenate([dst, jnp.asarray(n + pad_i % n_trash)])
    zeros_in = jnp.zeros((_CH, d), jnp.float32)

    sc_agg = _make_sc_agg(n, d, e_pad, n_pad)

    b1r = b1.reshape(1, d)
    b2r = b2.reshape(1, d)
    boutr = bout.reshape(1, -1)

    xt = _tc_pre0(h, W1, b1r, bn)
    p = sc_agg(src, dst, xt, zeros_in)
    xt = _tc_mid(p, W2, b2r, bn, n)
    p = sc_agg(src, dst, xt, zeros_in)
    return _tc_out(p, Wout, boutr, bn, n)


# speedup vs baseline: 3.3966x; 1.0028x over previous
import functools

import numpy as np

import jax
import jax.numpy as jnp
from jax import lax
from jax.experimental import pallas as pl
from jax.experimental.pallas import tpu as pltpu
from jax.experimental.pallas import tpu_sc as plsc

EPS = 1e-15


_MAXN = 1.0 - 1e-5


def _rowss(x):
    return jnp.sum(x * x, axis=-1).reshape(1, -1)


def _col(s):
    return s.reshape(-1, 1)


def _artanh(x):
    x = jnp.clip(x, -1 + 1e-7, 1 - 1e-7)
    return 0.5 * jnp.log((1 + x) / (1 - x))


def _expproj(ss):
    nu = jnp.clip(jnp.sqrt(ss), EPS, 1e15)
    t = jnp.tanh(nu)
    s = (t / nu) * jnp.minimum(1.0, _MAXN / jnp.clip(t, EPS, 1e15))
    nx = jnp.clip(jnp.minimum(t, _MAXN), EPS, 1e15)
    return s, nx


def _postagg_relu(agg):
    na = jnp.clip(jnp.sqrt(_rowss(agg)), EPS, 1e15)
    t = jnp.tanh(na)
    s_exp = t / na
    s_p = jnp.minimum(1.0, _MAXN / jnp.clip(t, EPS, 1e15))
    nh = jnp.clip(jnp.minimum(t, _MAXN), EPS, 1e15)
    s_lg = _artanh(nh) / nh
    return jax.nn.relu(agg * _col(s_exp * s_p * s_lg))


def _bias_row(b):
    nb = jnp.clip(jnp.sqrt(jnp.sum(b * b)), EPS, 1e15)
    e = b * (jnp.tanh(nb) / nb)
    ne = jnp.clip(jnp.sqrt(jnp.sum(e * e)), EPS, 1e15)
    bias = e * jnp.minimum(1.0, _MAXN / ne)
    return bias, jnp.sum(bias * bias)


def _hyplin_logmap(u, sx, nx, w, bias, y2):
    mm = lax.dot_general(
        u, w, (((1,), (1,)), ((), ())),
        preferred_element_type=jnp.float32, precision=lax.Precision.HIGHEST)
    ssmm = _rowss(mm)
    nmx = jnp.clip(jnp.sqrt(ssmm) * sx, EPS, 1e15)
    r = nmx / nx * _artanh(nx)
    tr = jnp.tanh(r)
    zero = ssmm == 0
    n_res = jnp.where(zero, 0.0, tr)
    s_mv = jnp.where(zero, 0.0, tr / nmx) * sx
    s_pmv = jnp.minimum(1.0, _MAXN / jnp.clip(n_res, EPS, 1e15))
    nmv = jnp.minimum(n_res, _MAXN)
    x2 = nmv * nmv
    s_all = s_mv * s_pmv
    xy = jnp.sum(mm * bias, axis=-1).reshape(1, -1) * s_all
    den = jnp.clip(1 + 2 * xy + x2 * y2, EPS, None)
    a = (1 + 2 * xy + y2) / den
    b2_ = (1 - x2) / den
    ssh = a * a * x2 + 2 * a * b2_ * xy + b2_ * b2_ * y2
    nhl = jnp.clip(jnp.sqrt(ssh), EPS, 1e15)
    s_ph = jnp.minimum(1.0, _MAXN / nhl)
    nhc = jnp.clip(jnp.minimum(nhl, _MAXN), EPS, 1e15)
    f = s_ph * (_artanh(nhc) / nhc)
    return mm * _col(s_all * a * f) + bias * _col(b2_ * f)


def _k_pre0(h_ref, w_ref, b_ref, o_ref):
    h = h_ref[...]
    sx, nx = _expproj(_rowss(h))
    bias, y2 = _bias_row(b_ref[...])
    o_ref[...] = _hyplin_logmap(h, sx, nx, w_ref[...], bias, y2)


def _k_mid(p_ref, w_ref, b_ref, o_ref):
    xt2 = _postagg_relu(p_ref[0] + p_ref[1])
    sx, nx = _expproj(_rowss(xt2))
    bias, y2 = _bias_row(b_ref[...])
    o_ref[...] = _hyplin_logmap(xt2, sx, nx, w_ref[...], bias, y2)


def _k_out(p_ref, wout_ref, bout_ref, o_ref):
    xt2 = _postagg_relu(p_ref[0] + p_ref[1])
    sx, _ = _expproj(_rowss(xt2))
    mm = lax.dot_general(
        xt2, wout_ref[...], (((1,), (1,)), ((), ())),
        preferred_element_type=jnp.float32, precision=lax.Precision.HIGHEST)
    o_ref[...] = mm * _col(sx) + bout_ref[...]


def _tc_pre0(h, w, b, bn):
    n, d = h.shape
    return pl.pallas_call(
        _k_pre0,
        out_shape=jax.ShapeDtypeStruct((n, d), jnp.float32),
        grid=(n // bn,),
        in_specs=[
            pl.BlockSpec((bn, d), lambda i: (i, 0)),
            pl.BlockSpec((d, d), lambda i: (0, 0)),
            pl.BlockSpec((1, d), lambda i: (0, 0)),
        ],
        out_specs=pl.BlockSpec((bn, d), lambda i: (i, 0)),
    )(h, w, b)


def _tc_mid(p, w, b, bn, n):
    d = p.shape[-1]
    return pl.pallas_call(
        _k_mid,
        out_shape=jax.ShapeDtypeStruct((n, d), jnp.float32),
        grid=(n // bn,),
        in_specs=[
            pl.BlockSpec((2, bn, d), lambda i: (0, i, 0)),
            pl.BlockSpec((d, d), lambda i: (0, 0)),
            pl.BlockSpec((1, d), lambda i: (0, 0)),
        ],
        out_specs=pl.BlockSpec((bn, d), lambda i: (i, 0)),
    )(p, w, b)


def _tc_out(p, wout, bout, bn, n):
    d = p.shape[-1]
    z = wout.shape[0]
    return pl.pallas_call(
        _k_out,
        out_shape=jax.ShapeDtypeStruct((n, z), jnp.float32),
        grid=(n // bn,),
        in_specs=[
            pl.BlockSpec((2, bn, d), lambda i: (0, i, 0)),
            pl.BlockSpec((z, d), lambda i: (0, 0)),
            pl.BlockSpec((1, z), lambda i: (0, 0)),
        ],
        out_specs=pl.BlockSpec((bn, z), lambda i: (i, 0)),
    )(p, wout, bout)


_CH = 128
_NC = 2
_NS = 16


_NBUF = 2
_NIDX = 4


@functools.cache
def _make_sc_agg(n, d, e_pad, n_pad):
    cpw = e_pad // (_NC * _NS * _CH)
    zch = n_pad // (_NS * _CH)
    outr = n_pad // _NS
    assert cpw % _NIDX == 0 and cpw >= 2 * _NIDX

    mesh = plsc.VectorSubcoreMesh(core_axis_name="c", subcore_axis_name="s")

    @functools.partial(
        pl.kernel,
        out_type=jax.ShapeDtypeStruct((_NC, n_pad, d), jnp.float32),
        mesh=mesh,
        scratch_types=[
            pltpu.VMEM_SHARED((n_pad, d), jnp.float32),
            [pltpu.VMEM((_CH,), jnp.int32)] * _NIDX,
            [pltpu.VMEM((_CH,), jnp.int32)] * _NIDX,
            [pltpu.VMEM((_CH, d), jnp.float32)] * _NBUF,
            [pltpu.SemaphoreType.DMA] * _NBUF,
            [pltpu.SemaphoreType.DMA] * _NBUF,
            [pltpu.SemaphoreType.DMA] * _NIDX,
            [pltpu.SemaphoreType.DMA] * _NIDX,
        ],
    )
    def agg_kernel(src_hbm, dst_hbm, xt_hbm, zeros_hbm, out_hbm,
                   acc_sh, idx_s, idx_d, rows, sem_g, sem_s, sem_is, sem_id):
        cid = lax.axis_index("c")
        sid = lax.axis_index("s")
        wid = cid * _NS + sid

        pltpu.sync_copy(zeros_hbm, rows[0])

        @pl.loop(0, zch)
        def _zero(k):
            pltpu.sync_copy(rows[0], acc_sh.at[pl.ds((sid * zch + k) * _CH, _CH)])

        plsc.subcore_barrier()

        def _idx(c, q):
            off = (c * _NC * _NS + wid) * _CH
            pltpu.async_copy(src_hbm.at[pl.ds(off, _CH)], idx_s[q], sem_is[q])
            pltpu.async_copy(dst_hbm.at[pl.ds(off, _CH)], idx_d[q], sem_id[q])

        def _gather(b, q):
            pltpu.async_copy(xt_hbm.at[idx_s[q]], rows[b], sem_g[b])

        def _scatter(b, q):
            pltpu.async_copy(rows[b], acc_sh.at[idx_d[q]], sem_s[b], add=True)

        def _drain_rows(sem):
            pltpu.make_async_copy(xt_hbm.at[pl.ds(0, _CH)], rows[0], sem).wait()

        def _drain_idx(sem):
            pltpu.make_async_copy(src_hbm.at[pl.ds(0, _CH)], idx_s[0],
                                  sem).wait()

        def _step_full(c, j):
            b, bp = j % _NBUF, (j - 1) % _NBUF
            q, qp, qn = j % _NIDX, (j - 1) % _NIDX, (j + 2) % _NIDX
            _drain_rows(sem_s[b])
            _idx(c + 2, qn)
            _drain_idx(sem_is[q])
            _gather(b, q)
            _drain_rows(sem_g[bp])
            _drain_idx(sem_id[qp])
            _scatter(bp, qp)

        _idx(0, 0)
        _idx(1, 1)
        _idx(2, 2)
        _drain_idx(sem_is[0])
        _gather(0, 0)
        _idx(3, 3)
        _drain_idx(sem_is[1])
        _gather(1, 1)
        _drain_rows(sem_g[0])
        _drain_idx(sem_id[0])
        _scatter(0, 0)
        _step_full(2, 2)
        _step_full(3, 3)

        @pl.loop(1, cpw // _NIDX)
        def _edges(g):
            for j in range(_NIDX):
                _step_full(g * _NIDX + j, j)

        _drain_rows(sem_g[(cpw - 1) % _NBUF])
        _drain_idx(sem_id[(cpw - 1) % _NIDX])
        _scatter((cpw - 1) % _NBUF, (cpw - 1) % _NIDX)
        for b in range(_NBUF):
            _drain_rows(sem_s[b])
        for q in (cpw % _NIDX, (cpw + 1) % _NIDX):
            _drain_idx(sem_is[q])
            _drain_idx(sem_id[q])

        plsc.subcore_barrier()
        pltpu.sync_copy(acc_sh.at[pl.ds(sid * outr, outr)],
                        out_hbm.at[cid, pl.ds(sid * outr, outr)])

    return agg_kernel


def kernel(h, distances, edges, node_mask, edge_mask, W1, b1, W2, b2, Wout, bout):
    n, d = h.shape
    e = edges.shape[1]

    bn = 2000 if n % 2000 == 0 else n

    chunk_tot = _NC * _NS * _CH * _NBUF
    e_pad = -(-e // chunk_tot) * chunk_tot
    n_pad = -(-(n + 1) // (_NS * _CH)) * (_NS * _CH)

    src = edges[0].astype(jnp.int32)
    dst = edges[1].astype(jnp.int32)
    pad = e_pad - e + 2 * _NC * _NS * _CH
    pad_i = np.arange(pad, dtype=np.int32)
    n_trash = n_pad - n
    src = jnp.concatenate([src, jnp.asarray(pad_i % n)])
    dst = jnp.concatenate([dst, jnp.asarray(n + pad_i % n_trash)])
    zeros_in = jnp.zeros((_CH, d), jnp.float32)

    sc_agg = _make_sc_agg(n, d, e_pad, n_pad)

    b1r = b1.reshape(1, d)
    b2r = b2.reshape(1, d)
    boutr = bout.reshape(1, -1)

    xt = _tc_pre0(h, W1, b1r, bn)
    p = sc_agg(src, dst, xt, zeros_in)
    xt = _tc_mid(p, W2, b2r, bn, n)
    p = sc_agg(src, dst, xt, zeros_in)
    return _tc_out(p, Wout, boutr, bn, n)
